# baseline (speedup 1.0000x reference)
import functools

import jax
import jax.numpy as jnp
from jax import lax
from jax.experimental import pallas as pl
from jax.experimental.pallas import tpu as pltpu
from jax.experimental.pallas import tpu_sc as plsc

_N = 10000
_E = 320000
_H = 128
_C = 40

_NC = 2
_NS = 16
_NW = _NC * _NS
_B = 80
_BE = 80
_RPS = _N // _NS

_NCH_E = _E // _NW // _BE
_NCH_C = _E // _NS // _B
_RING = 5


def _mesh():
    return plsc.VectorSubcoreMesh(core_axis_name="c", subcore_axis_name="s")


def _sc_agg_col(fw):
    @functools.partial(
        pl.kernel,
        mesh=_mesh(),
        compiler_params=pltpu.CompilerParams(use_tc_tiling_on_sc=False),
        out_type=jax.ShapeDtypeStruct((_NC, _N, fw), jnp.float32),
        scratch_types=[
            pltpu.VMEM((_NCH_C, _B), jnp.int32),
            pltpu.VMEM((_NCH_C, _B), jnp.int32),
            [pltpu.VMEM((_B, fw), jnp.float32) for _ in range(_RING)],
            [pltpu.SemaphoreType.DMA for _ in range(_RING)],
            pltpu.VMEM_SHARED((_N, fw), jnp.float32),
        ],
    )
    def k(y_hbm, edges_hbm, zeros_hbm, out_hbm,
          src_v, dst_v, bufs, sems, acc):
        c = lax.axis_index("c")
        s = lax.axis_index("s")
        coff = c * _N

        pltpu.sync_copy(edges_hbm.at[s], src_v)
        pltpu.sync_copy(zeros_hbm, acc.at[pl.ds(s * _RPS, _RPS)])

        def unpack(r, carry):
            for l in range(_B // 16):
                sl = pl.ds(l * 16, 16)
                p = src_v[r, sl]
                dst_v[r, sl] = lax.shift_right_logical(p, 16)
                src_v[r, sl] = (p & 0xFFFF) + coff
            return carry

        lax.fori_loop(0, _NCH_C, unpack, 0, unroll=False)
        plsc.subcore_barrier()

        for b in range(_RING):
            pltpu.async_copy(y_hbm.at[src_v.at[b]], bufs[b], sems[b])

        def body(g, carry):
            j0 = g * _RING
            for b in range(_RING):
                j = j0 + b
                pltpu.make_async_copy(y_hbm.at[src_v.at[j]], bufs[b],
                                      sems[b]).wait()
                pltpu.sync_copy(bufs[b], acc.at[dst_v.at[j]], add=True)
                pltpu.async_copy(y_hbm.at[src_v.at[j + _RING]], bufs[b],
                                 sems[b])
            return carry

        lax.fori_loop(0, _NCH_C // _RING - 1, body, 0, unroll=False)
        for b in range(_RING):
            j = _NCH_C - _RING + b
            pltpu.make_async_copy(y_hbm.at[src_v.at[j]], bufs[b],
                                  sems[b]).wait()
            pltpu.sync_copy(bufs[b], acc.at[dst_v.at[j]], add=True)

        plsc.subcore_barrier()
        pltpu.sync_copy(acc.at[pl.ds(s * _RPS, _RPS)],
                        out_hbm.at[c].at[pl.ds(s * _RPS, _RPS)])

    return k


def _sc_agg_edge(fw, nbuf):
    @functools.partial(
        pl.kernel,
        mesh=_mesh(),
        compiler_params=pltpu.CompilerParams(use_tc_tiling_on_sc=False),
        out_type=jax.ShapeDtypeStruct((_NC, _N, fw), jnp.float32),
        scratch_types=[
            pltpu.VMEM((_NCH_E, _BE), jnp.int32),
            pltpu.VMEM((_NCH_E, _BE), jnp.int32),
            [pltpu.VMEM((_BE, fw), jnp.float32) for _ in range(nbuf)],
            [pltpu.SemaphoreType.DMA for _ in range(nbuf)],
            pltpu.VMEM_SHARED((_N, fw), jnp.float32),
        ],
    )
    def k(y_hbm, edges_hbm, zeros_hbm, out_hbm,
          src_v, dst_v, bufs, sems, acc):
        c = lax.axis_index("c")
        s = lax.axis_index("s")
        wid = s * _NC + c

        pltpu.sync_copy(edges_hbm.at[wid], src_v)
        pltpu.sync_copy(zeros_hbm, acc.at[pl.ds(s * _RPS, _RPS)])

        def unpack(r, carry):
            for l in range(_BE // 16):
                sl = pl.ds(l * 16, 16)
                p = src_v[r, sl]
                dst_v[r, sl] = lax.shift_right_logical(p, 16)
                src_v[r, sl] = p & 0xFFFF
            return carry

        lax.fori_loop(0, _NCH_E, unpack, 0, unroll=False)
        plsc.subcore_barrier()

        if nbuf == 2:
            b0, b1 = bufs
            s0, s1 = sems
            pltpu.async_copy(y_hbm.at[src_v.at[0]], b0, s0)

            def body(jj, carry):
                j0 = jj * 2
                pltpu.async_copy(y_hbm.at[src_v.at[j0 + 1]], b1, s1)
                pltpu.make_async_copy(y_hbm.at[src_v.at[j0]], b0, s0).wait()
                pltpu.sync_copy(b0, acc.at[dst_v.at[j0]], add=True)
                pltpu.async_copy(y_hbm.at[src_v.at[j0 + 2]], b0, s0)
                pltpu.make_async_copy(y_hbm.at[src_v.at[j0 + 1]], b1,
                                      s1).wait()
                pltpu.sync_copy(b1, acc.at[dst_v.at[j0 + 1]], add=True)
                return carry

            lax.fori_loop(0, _NCH_E // 2, body, 0, unroll=False)
            pltpu.make_async_copy(y_hbm.at[src_v.at[_NCH_E - 1]], b0,
                                  s0).wait()
            pltpu.sync_copy(b0, acc.at[dst_v.at[_NCH_E - 1]], add=True)
        else:
            for b in range(nbuf):
                pltpu.async_copy(y_hbm.at[src_v.at[b]], bufs[b], sems[b])

            def body(g, carry):
                j0 = g * nbuf
                for b in range(nbuf):
                    j = j0 + b
                    pltpu.make_async_copy(y_hbm.at[src_v.at[j]], bufs[b],
                                          sems[b]).wait()
                    pltpu.sync_copy(bufs[b], acc.at[dst_v.at[j]], add=True)
                    pltpu.async_copy(y_hbm.at[src_v.at[j + nbuf]], bufs[b],
                                     sems[b])
                return carry

            lax.fori_loop(0, _NCH_E // nbuf - 1, body, 0, unroll=False)
            for b in range(nbuf):
                j = _NCH_E - nbuf + b
                pltpu.make_async_copy(y_hbm.at[src_v.at[j]], bufs[b],
                                      sems[b]).wait()
                pltpu.sync_copy(bufs[b], acc.at[dst_v.at[j]], add=True)

        plsc.subcore_barrier()
        pltpu.sync_copy(acc.at[pl.ds(s * _RPS, _RPS)],
                        out_hbm.at[c].at[pl.ds(s * _RPS, _RPS)])

    return k


def _dense_in(x, wl, bl, wr):
    def body(x_ref, wl_ref, bl_ref, wr_ref, y_ref, z_ref):
        xv = x_ref[...]
        dn = (((1,), (1,)), ((), ()))
        m = lax.dot_general(xv, wl_ref[...], dn,
                            preferred_element_type=jnp.float32)
        y_ref[:_N, :] = m[:, :80]
        y_ref[_N:, :48] = m[:, 80:]
        col = lax.broadcasted_iota(jnp.int32, (_N, 32), 1)
        y_ref[_N:, 48:] = jnp.where(col == 0, 1.0, 0.0)
        z_ref[...] = lax.dot_general(xv, wr_ref[...], dn,
                                     preferred_element_type=jnp.float32) \
            + bl_ref[...][None, :]

    return pl.pallas_call(
        body,
        out_shape=[jax.ShapeDtypeStruct((2 * _N, 80), jnp.float32),
                   jax.ShapeDtypeStruct((_N, _H), jnp.float32)],
    )(x, wl, bl, wr)


def _dense_mid(p, z_prev, wl, bl, wr):
    def body(p_ref, zp_ref, wl_ref, bl_ref, wr_ref, y_ref, z_ref, r_ref):
        cnt = p_ref[1, :, 48:49]
        rcnt = 1.0 / jnp.maximum(cnt, 1.0)
        svec = jnp.concatenate([p_ref[0], p_ref[1, :, :48]], axis=1)
        h = jnp.maximum(svec * rcnt + zp_ref[...], 0.0)
        dn = (((1,), (1,)), ((), ()))
        m = lax.dot_general(h, wl_ref[...], dn,
                            preferred_element_type=jnp.float32)
        y_ref[:_N, :] = m[:, :64]
        y_ref[_N:, :] = m[:, 64:]
        z_ref[...] = lax.dot_general(h, wr_ref[...], dn,
                                     preferred_element_type=jnp.float32) \
            + bl_ref[...][None, :]
        r_ref[...] = rcnt

    return pl.pallas_call(
        body,
        out_shape=[jax.ShapeDtypeStruct((2 * _N, 64), jnp.float32),
                   jax.ShapeDtypeStruct((_N, _H), jnp.float32),
                   jax.ShapeDtypeStruct((_N, 1), jnp.float32)],
    )(p, z_prev, wl, bl, wr)


def _dense_out(p, rcnt, z_prev, wl, bl, wr):
    def body(p_ref, r_ref, zp_ref, wl_ref, bl_ref, wr_ref, y_ref, z_ref):
        svec = jnp.concatenate([p_ref[0], p_ref[1]], axis=1)
        h = jnp.maximum(svec * r_ref[...] + zp_ref[...], 0.0)
        dn = (((1,), (1,)), ((), ()))
        y_ref[:, :_C] = lax.dot_general(h, wl_ref[...], dn,
                                        preferred_element_type=jnp.float32)
        y_ref[:, _C:] = jnp.zeros((y_ref.shape[0], 48 - _C), jnp.float32)
        z_ref[...] = lax.dot_general(h, wr_ref[...], dn,
                                     preferred_element_type=jnp.float32) \
            + bl_ref[...][None, :]

    grid = 10
    blk = _N // grid
    return pl.pallas_call(
        body,
        grid=(grid,),
        in_specs=[
            pl.BlockSpec((_NC, blk, 64), lambda i: (0, i, 0)),
            pl.BlockSpec((blk, 1), lambda i: (i, 0)),
            pl.BlockSpec((blk, _H), lambda i: (i, 0)),
            pl.BlockSpec((_C, _H), lambda i: (0, 0)),
            pl.BlockSpec((_C,), lambda i: (0,)),
            pl.BlockSpec((_C, _H), lambda i: (0, 0)),
        ],
        out_specs=[
            pl.BlockSpec((blk, 48), lambda i: (i, 0)),
            pl.BlockSpec((blk, _C), lambda i: (i, 0)),
        ],
        out_shape=[jax.ShapeDtypeStruct((_N, 48), jnp.float32),
                   jax.ShapeDtypeStruct((_N, _C), jnp.float32)],
    )(p, rcnt, z_prev, wl, bl, wr)


def _final(p, rcnt, z_prev):
    def body(p_ref, r_ref, zp_ref, o_ref):
        svec = p_ref[0] + p_ref[1]
        o_ref[...] = svec[:, :_C] * r_ref[...] + zp_ref[...]

    grid = 10
    blk = _N // grid
    return pl.pallas_call(
        body,
        grid=(grid,),
        in_specs=[
            pl.BlockSpec((_NC, blk, 48), lambda i: (0, i, 0)),
            pl.BlockSpec((blk, 1), lambda i: (i, 0)),
            pl.BlockSpec((blk, _C), lambda i: (i, 0)),
        ],
        out_specs=pl.BlockSpec((blk, _C), lambda i: (i, 0)),
        out_shape=jax.ShapeDtypeStruct((_N, _C), jnp.float32),
    )(p, rcnt, z_prev)


def kernel(x, edge_index, Wl1, bl1, Wr1, Wl2, bl2, Wr2, Wl3, bl3, Wr3):
    packed = edge_index[0] | (edge_index[1] << 16)
    pk_col = packed.reshape(_NS, _NCH_C, _B)
    pk_edge = packed.reshape(_NW, _NCH_E, _BE)
    z80 = jnp.zeros((_RPS, 80), jnp.float32)
    z64 = jnp.zeros((_RPS, 64), jnp.float32)
    z48 = jnp.zeros((_RPS, 48), jnp.float32)

    y1, zr1 = _dense_in(x, Wl1, bl1, Wr1)
    p1 = _sc_agg_col(80)(y1, pk_col, z80)
    y2, zr2, rcnt = _dense_mid(p1, zr1, Wl2, bl2, Wr2)
    p2 = _sc_agg_col(64)(y2, pk_col, z64)
    y3, zr3 = _dense_out(p2, rcnt, zr2, Wl3, bl3, Wr3)
    p3 = _sc_agg_edge(48, _RING)(y3, pk_edge, z48)
    return _final(p3, rcnt, zr3)


# speedup vs baseline: 1.0065x; 1.0065x over previous
import functools

import jax
import jax.numpy as jnp
from jax import lax
from jax.experimental import pallas as pl
from jax.experimental.pallas import tpu as pltpu
from jax.experimental.pallas import tpu_sc as plsc

_N = 10000
_E = 320000
_H = 128
_C = 40

_NC = 2
_NS = 16
_NW = _NC * _NS
_B = 80
_BE = 80
_RPS = _N // _NS

_NCH_E = _E // _NW // _BE
_NCH_C = _E // _NS // _B
_RING = 5


def _mesh():
    return plsc.VectorSubcoreMesh(core_axis_name="c", subcore_axis_name="s")


def _sc_agg_col(fw):
    @functools.partial(
        pl.kernel,
        mesh=_mesh(),
        compiler_params=pltpu.CompilerParams(use_tc_tiling_on_sc=False),
        out_type=jax.ShapeDtypeStruct((_NC, _N, fw), jnp.float32),
        scratch_types=[
            pltpu.VMEM((_NCH_C, _B), jnp.int32),
            pltpu.VMEM((_NCH_C, _B), jnp.int32),
            [pltpu.VMEM((_B, fw), jnp.float32) for _ in range(_RING)],
            [pltpu.SemaphoreType.DMA for _ in range(_RING)],
            pltpu.VMEM_SHARED((_N, fw), jnp.float32),
        ],
    )
    def k(y_hbm, edges_hbm, zeros_hbm, out_hbm,
          src_v, dst_v, bufs, sems, acc):
        c = lax.axis_index("c")
        s = lax.axis_index("s")
        coff = c * _N

        pltpu.sync_copy(edges_hbm.at[s], src_v)
        pltpu.sync_copy(zeros_hbm, acc.at[pl.ds(s * _RPS, _RPS)])

        def unpack(r, carry):
            for l in range(_B // 16):
                sl = pl.ds(l * 16, 16)
                p = src_v[r, sl]
                dst_v[r, sl] = lax.shift_right_logical(p, 16)
                src_v[r, sl] = (p & 0xFFFF) + coff
            return carry

        lax.fori_loop(0, _NCH_C, unpack, 0, unroll=False)
        plsc.subcore_barrier()

        for b in range(_RING):
            pltpu.async_copy(y_hbm.at[src_v.at[b]], bufs[b], sems[b])

        def body(g, carry):
            j0 = g * _RING
            for b in range(_RING):
                j = j0 + b
                pltpu.make_async_copy(y_hbm.at[src_v.at[j]], bufs[b],
                                      sems[b]).wait()
                pltpu.sync_copy(bufs[b], acc.at[dst_v.at[j]], add=True)
                pltpu.async_copy(y_hbm.at[src_v.at[j + _RING]], bufs[b],
                                 sems[b])
            return carry

        lax.fori_loop(0, _NCH_C // _RING - 1, body, 0, unroll=False)
        for b in range(_RING):
            j = _NCH_C - _RING + b
            pltpu.make_async_copy(y_hbm.at[src_v.at[j]], bufs[b],
                                  sems[b]).wait()
            pltpu.sync_copy(bufs[b], acc.at[dst_v.at[j]], add=True)

        plsc.subcore_barrier()
        pltpu.sync_copy(acc.at[pl.ds(s * _RPS, _RPS)],
                        out_hbm.at[c].at[pl.ds(s * _RPS, _RPS)])

    return k


def _sc_agg_edge(fw, nbuf):
    @functools.partial(
        pl.kernel,
        mesh=_mesh(),
        compiler_params=pltpu.CompilerParams(use_tc_tiling_on_sc=False),
        out_type=jax.ShapeDtypeStruct((_NC, _N, fw), jnp.float32),
        scratch_types=[
            pltpu.VMEM((_NCH_E, _BE), jnp.int32),
            pltpu.VMEM((_NCH_E, _BE), jnp.int32),
            [pltpu.VMEM((_BE, fw), jnp.float32) for _ in range(nbuf)],
            [pltpu.SemaphoreType.DMA for _ in range(nbuf)],
            pltpu.VMEM_SHARED((_N, fw), jnp.float32),
        ],
    )
    def k(y_hbm, edges_hbm, zeros_hbm, out_hbm,
          src_v, dst_v, bufs, sems, acc):
        c = lax.axis_index("c")
        s = lax.axis_index("s")
        wid = s * _NC + c

        pltpu.sync_copy(edges_hbm.at[wid], src_v)
        pltpu.sync_copy(zeros_hbm, acc.at[pl.ds(s * _RPS, _RPS)])

        def unpack(r, carry):
            for l in range(_BE // 16):
                sl = pl.ds(l * 16, 16)
                p = src_v[r, sl]
                dst_v[r, sl] = lax.shift_right_logical(p, 16)
                src_v[r, sl] = p & 0xFFFF
            return carry

        lax.fori_loop(0, _NCH_E, unpack, 0, unroll=False)
        plsc.subcore_barrier()

        if nbuf == 2:
            b0, b1 = bufs
            s0, s1 = sems
            pltpu.async_copy(y_hbm.at[src_v.at[0]], b0, s0)

            def body(jj, carry):
                j0 = jj * 2
                pltpu.async_copy(y_hbm.at[src_v.at[j0 + 1]], b1, s1)
                pltpu.make_async_copy(y_hbm.at[src_v.at[j0]], b0, s0).wait()
                pltpu.sync_copy(b0, acc.at[dst_v.at[j0]], add=True)
                pltpu.async_copy(y_hbm.at[src_v.at[j0 + 2]], b0, s0)
                pltpu.make_async_copy(y_hbm.at[src_v.at[j0 + 1]], b1,
                                      s1).wait()
                pltpu.sync_copy(b1, acc.at[dst_v.at[j0 + 1]], add=True)
                return carry

            lax.fori_loop(0, _NCH_E // 2, body, 0, unroll=False)
            pltpu.make_async_copy(y_hbm.at[src_v.at[_NCH_E - 1]], b0,
                                  s0).wait()
            pltpu.sync_copy(b0, acc.at[dst_v.at[_NCH_E - 1]], add=True)
        else:
            for b in range(nbuf):
                pltpu.async_copy(y_hbm.at[src_v.at[b]], bufs[b], sems[b])

            def body(g, carry):
                j0 = g * nbuf
                for b in range(nbuf):
                    j = j0 + b
                    pltpu.make_async_copy(y_hbm.at[src_v.at[j]], bufs[b],
                                          sems[b]).wait()
                    pltpu.sync_copy(bufs[b], acc.at[dst_v.at[j]], add=True)
                    pltpu.async_copy(y_hbm.at[src_v.at[j + nbuf]], bufs[b],
                                     sems[b])
                return carry

            lax.fori_loop(0, _NCH_E // nbuf - 1, body, 0, unroll=False)
            for b in range(nbuf):
                j = _NCH_E - nbuf + b
                pltpu.make_async_copy(y_hbm.at[src_v.at[j]], bufs[b],
                                      sems[b]).wait()
                pltpu.sync_copy(bufs[b], acc.at[dst_v.at[j]], add=True)

        plsc.subcore_barrier()
        pltpu.sync_copy(acc.at[pl.ds(s * _RPS, _RPS)],
                        out_hbm.at[c].at[pl.ds(s * _RPS, _RPS)])

    return k


def _dense_in(x, wl, bl, wr):
    def body(x_ref, wl_ref, bl_ref, wr_ref, y_ref, z_ref):
        xv = x_ref[...]
        dn = (((1,), (1,)), ((), ()))
        m = lax.dot_general(xv, wl_ref[...], dn,
                            preferred_element_type=jnp.float32)
        y_ref[:_N, :] = m[:, :80]
        y_ref[_N:, :48] = m[:, 80:]
        col = lax.broadcasted_iota(jnp.int32, (_N, 32), 1)
        y_ref[_N:, 48:] = jnp.where(col == 0, 1.0, 0.0)
        z_ref[...] = lax.dot_general(xv, wr_ref[...], dn,
                                     preferred_element_type=jnp.float32) \
            + bl_ref[...][None, :]

    return pl.pallas_call(
        body,
        out_shape=[jax.ShapeDtypeStruct((2 * _N, 80), jnp.float32),
                   jax.ShapeDtypeStruct((_N, _H), jnp.float32)],
    )(x, wl, bl, wr)


def _dense_mid(p, z_prev, wl, bl, wr):
    def body(p_ref, zp_ref, wl_ref, bl_ref, wr_ref, y_ref, z_ref, r_ref):
        cnt = p_ref[1, :, 48:49]
        rcnt = 1.0 / jnp.maximum(cnt, 1.0)
        svec = jnp.concatenate([p_ref[0], p_ref[1, :, :48]], axis=1)
        h = jnp.maximum(svec * rcnt + zp_ref[...], 0.0)
        dn = (((1,), (1,)), ((), ()))
        m = lax.dot_general(h, wl_ref[...], dn,
                            preferred_element_type=jnp.float32)
        y_ref[:_N, :] = m[:, :64]
        y_ref[_N:, :] = m[:, 64:]
        z_ref[...] = lax.dot_general(h, wr_ref[...], dn,
                                     preferred_element_type=jnp.float32) \
            + bl_ref[...][None, :]
        r_ref[...] = rcnt

    return pl.pallas_call(
        body,
        out_shape=[jax.ShapeDtypeStruct((2 * _N, 64), jnp.float32),
                   jax.ShapeDtypeStruct((_N, _H), jnp.float32),
                   jax.ShapeDtypeStruct((_N, 1), jnp.float32)],
    )(p, z_prev, wl, bl, wr)


def _dense_out(p, rcnt, z_prev, wl, bl, wr):
    def body(p_ref, r_ref, zp_ref, wl_ref, bl_ref, wr_ref, y_ref, z_ref):
        svec = jnp.concatenate([p_ref[0], p_ref[1]], axis=1)
        h = jnp.maximum(svec * r_ref[...] + zp_ref[...], 0.0)
        dn = (((1,), (1,)), ((), ()))
        y_ref[:, :_C] = lax.dot_general(h, wl_ref[...], dn,
                                        preferred_element_type=jnp.float32)
        y_ref[:, _C:] = jnp.zeros((_N, 48 - _C), jnp.float32)
        z_ref[...] = lax.dot_general(h, wr_ref[...], dn,
                                     preferred_element_type=jnp.float32) \
            + bl_ref[...][None, :]

    return pl.pallas_call(
        body,
        out_shape=[jax.ShapeDtypeStruct((_N, 48), jnp.float32),
                   jax.ShapeDtypeStruct((_N, _C), jnp.float32)],
    )(p, rcnt, z_prev, wl, bl, wr)


def _final(p, rcnt, z_prev):
    def body(p_ref, r_ref, zp_ref, o_ref):
        svec = p_ref[0] + p_ref[1]
        o_ref[...] = svec[:, :_C] * r_ref[...] + zp_ref[...]

    return pl.pallas_call(
        body,
        out_shape=jax.ShapeDtypeStruct((_N, _C), jnp.float32),
    )(p, rcnt, z_prev)


def kernel(x, edge_index, Wl1, bl1, Wr1, Wl2, bl2, Wr2, Wl3, bl3, Wr3):
    packed = edge_index[0] | (edge_index[1] << 16)
    pk_col = packed.reshape(_NS, _NCH_C, _B)
    pk_edge = packed.reshape(_NW, _NCH_E, _BE)
    z80 = jnp.zeros((_RPS, 80), jnp.float32)
    z64 = jnp.zeros((_RPS, 64), jnp.float32)
    z48 = jnp.zeros((_RPS, 48), jnp.float32)

    y1, zr1 = _dense_in(x, Wl1, bl1, Wr1)
    p1 = _sc_agg_col(80)(y1, pk_col, z80)
    y2, zr2, rcnt = _dense_mid(p1, zr1, Wl2, bl2, Wr2)
    p2 = _sc_agg_col(64)(y2, pk_col, z64)
    y3, zr3 = _dense_out(p2, rcnt, zr2, Wl3, bl3, Wr3)
    p3 = _sc_agg_edge(48, _RING)(y3, pk_edge, z48)
    return _final(p3, rcnt, zr3)
